# baseline (speedup 1.0000x reference)
import functools

import jax
import jax.numpy as jnp
from jax import lax
from jax.experimental import pallas as pl
from jax.experimental.pallas import tpu as pltpu
from jax.experimental.pallas import tpu_sc as plsc

BATCH = 4096
HIST = 50
EMBED = 32
WINDOW = 800
NBUF = 3


@functools.lru_cache(maxsize=None)
def _make_gather():
    info = plsc.get_sparse_core_info()
    nc, ns, nl = info.num_cores, info.num_subcores, info.num_lanes
    nw = nc * ns
    total = BATCH * HIST
    b_per_w = total // nw
    n_win = b_per_w // WINDOW
    vecs_per_win = WINDOW // nl

    mesh = plsc.VectorSubcoreMesh(core_axis_name="c", subcore_axis_name="s")

    @functools.partial(
        pl.kernel,
        mesh=mesh,
        out_type=jax.ShapeDtypeStruct((total, EMBED), jnp.float32),
        scratch_types=[
            pltpu.VMEM((b_per_w // nl, nl), jnp.int32),
            pltpu.VMEM((NBUF, WINDOW, EMBED), jnp.float32),
            pltpu.SemaphoreType.DMA,
            pltpu.SemaphoreType.DMA((NBUF,)),
        ],
        compiler_params=pltpu.CompilerParams(use_tc_tiling_on_sc=False),
    )
    def gather_kernel(idx_hbm, table_hbm, out_hbm, idx_v, rows, gsem, ssem):
        wid = lax.axis_index("s") * nc + lax.axis_index("c")
        base = wid * b_per_w
        pltpu.sync_copy(idx_hbm.at[wid], idx_v)

        def enqueue_window(w):
            b = w % NBUF
            for k in range(vecs_per_win):
                vec = idx_v[w * vecs_per_win + k]
                pltpu.async_copy(
                    table_hbm.at[vec], rows.at[b, pl.ds(k * nl, nl)], gsem
                )

        def drain_window():
            pltpu.make_async_copy(
                table_hbm.at[pl.ds(0, WINDOW)], rows.at[0], gsem
            ).wait()

        def out_slot(w):
            return out_hbm.at[pl.ds(base + w * WINDOW, WINDOW)]

        def store_start(w):
            b = w % NBUF
            pltpu.async_copy(rows.at[b], out_slot(w), ssem.at[b])

        def store_wait(w):
            b = w % NBUF
            pltpu.make_async_copy(rows.at[b], out_slot(w), ssem.at[b]).wait()

        enqueue_window(0)
        for w in range(1, n_win):
            if w >= NBUF:
                store_wait(w - NBUF)
            enqueue_window(w)
            drain_window()
            store_start(w - 1)
        drain_window()
        store_start(n_win - 1)
        for w in range(n_win - NBUF, n_win):
            store_wait(w)

    return gather_kernel, nw, b_per_w // nl, nl


def kernel(x, table):
    gather_fn, nw, vecs, nl = _make_gather()
    idx = x.reshape(nw, vecs, nl).astype(jnp.int32)
    out = gather_fn(idx, table)
    return out.reshape(BATCH, HIST, EMBED)


# speedup vs baseline: 1.0011x; 1.0011x over previous
import functools

import jax
import jax.numpy as jnp
from jax import lax
from jax.experimental import pallas as pl
from jax.experimental.pallas import tpu as pltpu
from jax.experimental.pallas import tpu_sc as plsc

BATCH = 4096
HIST = 50
EMBED = 32
WINDOW = 800
NBUF = 3


@functools.lru_cache(maxsize=None)
def _make_gather():
    info = plsc.get_sparse_core_info()
    nc, ns, nl = info.num_cores, info.num_subcores, info.num_lanes
    nw = nc * ns
    total = BATCH * HIST
    b_per_w = total // nw
    n_win = b_per_w // WINDOW

    mesh = plsc.VectorSubcoreMesh(core_axis_name="c", subcore_axis_name="s")

    @functools.partial(
        pl.kernel,
        mesh=mesh,
        out_type=jax.ShapeDtypeStruct((total, EMBED), jnp.float32),
        scratch_types=[
            pltpu.VMEM((b_per_w,), jnp.int32),
            pltpu.VMEM((NBUF, WINDOW, EMBED), jnp.float32),
            pltpu.SemaphoreType.DMA((NBUF,)),
            pltpu.SemaphoreType.DMA((NBUF,)),
        ],
        compiler_params=pltpu.CompilerParams(use_tc_tiling_on_sc=False),
    )
    def gather_kernel(idx_hbm, table_hbm, out_hbm, idx_v, rows, gsem, ssem):
        wid = lax.axis_index("s") * nc + lax.axis_index("c")
        base = wid * b_per_w
        pltpu.sync_copy(idx_hbm.at[pl.ds(base, b_per_w)], idx_v)

        def gather_start(w):
            return pltpu.async_copy(
                table_hbm.at[idx_v.at[pl.ds(w * WINDOW, WINDOW)]],
                rows.at[w % NBUF],
                gsem.at[w % NBUF],
            )

        def store_start(w):
            return pltpu.async_copy(
                rows.at[w % NBUF],
                out_hbm.at[pl.ds(base + w * WINDOW, WINDOW)],
                ssem.at[w % NBUF],
            )

        gh = [None] * n_win
        sh = [None] * n_win
        for w in range(min(NBUF, n_win)):
            gh[w] = gather_start(w)
        for w in range(n_win):
            gh[w].wait()
            sh[w] = store_start(w)
            nxt = w + NBUF
            if nxt < n_win:
                sh[w].wait()
                gh[nxt] = gather_start(nxt)
        for w in range(n_win - NBUF, n_win):
            sh[w].wait()

    return gather_kernel, nw


def kernel(x, table):
    gather_fn, nw = _make_gather()
    idx = x.reshape(-1).astype(jnp.int32)
    out = gather_fn(idx, table)
    return out.reshape(BATCH, HIST, EMBED)


# speedup vs baseline: 1.0030x; 1.0019x over previous
import functools

import jax
import jax.numpy as jnp
from jax import lax
from jax.experimental import pallas as pl
from jax.experimental.pallas import tpu as pltpu
from jax.experimental.pallas import tpu_sc as plsc

BATCH = 4096
HIST = 50
EMBED = 32
WINDOW = 800
NBUF = 3


@functools.lru_cache(maxsize=None)
def _make_gather():
    info = plsc.get_sparse_core_info()
    nc, ns, nl = info.num_cores, info.num_subcores, info.num_lanes
    nw = nc * ns
    total = BATCH * HIST
    b_per_w = total // nw
    n_win = b_per_w // WINDOW

    mesh = plsc.VectorSubcoreMesh(core_axis_name="c", subcore_axis_name="s")

    @functools.partial(
        pl.kernel,
        mesh=mesh,
        out_type=jax.ShapeDtypeStruct((total, EMBED), jnp.float32),
        scratch_types=[
            pltpu.VMEM((b_per_w,), jnp.int32),
            pltpu.VMEM((NBUF, WINDOW, EMBED), jnp.float32),
            pltpu.SemaphoreType.DMA,
            pltpu.SemaphoreType.DMA((NBUF,)),
        ],
        compiler_params=pltpu.CompilerParams(use_tc_tiling_on_sc=False),
    )
    def gather_kernel(idx_hbm, table_hbm, out_hbm, idx_v, rows, gsem, ssem):
        wid = lax.axis_index("s") * nc + lax.axis_index("c")
        base = wid * b_per_w
        pltpu.sync_copy(idx_hbm.at[pl.ds(base, b_per_w)], idx_v)

        def enqueue_window(w):
            b = w % NBUF
            pltpu.async_copy(
                table_hbm.at[idx_v.at[pl.ds(w * WINDOW, WINDOW)]],
                rows.at[b],
                gsem,
            )

        def drain_window():
            pltpu.make_async_copy(
                table_hbm.at[pl.ds(0, WINDOW)], rows.at[0], gsem
            ).wait()

        def out_slot(w):
            return out_hbm.at[pl.ds(base + w * WINDOW, WINDOW)]

        def store_start(w):
            b = w % NBUF
            pltpu.async_copy(rows.at[b], out_slot(w), ssem.at[b])

        def store_wait(w):
            b = w % NBUF
            pltpu.make_async_copy(rows.at[b], out_slot(w), ssem.at[b]).wait()

        enqueue_window(0)
        for w in range(1, n_win):
            if w >= NBUF:
                store_wait(w - NBUF)
            enqueue_window(w)
            drain_window()
            store_start(w - 1)
        drain_window()
        store_start(n_win - 1)
        for w in range(n_win - NBUF, n_win):
            store_wait(w)

    return gather_kernel, nw


def kernel(x, table):
    gather_fn, nw = _make_gather()
    idx = x.reshape(-1).astype(jnp.int32)
    out = gather_fn(idx, table)
    return out.reshape(BATCH, HIST, EMBED)
